# baseline (speedup 1.0000x reference)
import functools

import jax
import jax.numpy as jnp
import numpy as np
from jax import lax
from jax.experimental import pallas as pl
from jax.experimental.pallas import tpu as pltpu
from jax.experimental.pallas import tpu_sc as plsc

_N = 10000
_E = 320000
_D = 128
_DE = 16
_NC = 2
_NS = 16
_NT = _NC * _NS
_EPT = _E // _NT
_C = 40
_NCH = _EPT // _C
_RPT = 640
_ZC = 40

_PERM = np.concatenate(
    [np.concatenate([32 * j + 2 * np.arange(16),
                     32 * j + 2 * np.arange(16) + 1]) for j in range(4)]
)


def _tc_edge_embed(edge_attr, W_edge, b_edge):
    B = 3200

    def body(a_ref, w_ref, b_ref, o_ref):
        acc = (
            jnp.dot(a_ref[...], w_ref[...], preferred_element_type=jnp.float32)
            + b_ref[...]
        )
        o_ref[...] = acc.astype(jnp.bfloat16)

    return pl.pallas_call(
        body,
        grid=(_E // B,),
        in_specs=[
            pl.BlockSpec((B, _DE), lambda i: (i, 0)),
            pl.BlockSpec((_DE, _D), lambda i: (0, 0)),
            pl.BlockSpec((1, _D), lambda i: (0, 0)),
        ],
        out_specs=pl.BlockSpec((B, _D), lambda i: (i, 0)),
        out_shape=jax.ShapeDtypeStruct((_E, _D), jnp.bfloat16),
    )(edge_attr, W_edge, b_edge.reshape(1, _D))


def _sc_edge_aggr(x, ids, e):
    mesh = plsc.VectorSubcoreMesh(core_axis_name="c", subcore_axis_name="s")

    @functools.partial(
        pl.kernel,
        out_type=jax.ShapeDtypeStruct((_NC, _N, _D), jnp.float32),
        mesh=mesh,
        scratch_types=[
            pltpu.VMEM_SHARED((_N, _D), jnp.float32),
            pltpu.VMEM((4, 2, _C), jnp.int32),
            pltpu.VMEM((2, _C, _D), jnp.bfloat16),
            pltpu.VMEM((2, _C, _D), jnp.bfloat16),
            pltpu.VMEM((2, _C, _D), jnp.float32),
            pltpu.SemaphoreType.DMA((4,)),
            pltpu.SemaphoreType.DMA((2,)),
            pltpu.SemaphoreType.DMA((2,)),
            pltpu.SemaphoreType.DMA((2,)),
        ],
        compiler_params=pltpu.CompilerParams(use_tc_tiling_on_sc=False,
                                             needs_layout_passes=False),
    )
    def k(x_hbm, ids_hbm, e_hbm, out_hbm,
          aggr, idr, xg, eb, mb, isem, gsem, esem, ssem):
        c = lax.axis_index("c")
        s = lax.axis_index("s")
        tid = c * _NS + s
        ebase = tid * _EPT

        nz = jnp.where(s == _NS - 1, (_N - (_NS - 1) * _RPT) // _ZC, _RPT // _ZC)

        def zrow(i, _):
            for j in range(_D // 16):
                mb[0, i, pl.ds(j * 16, 16)] = jnp.zeros((16,), jnp.float32)
            return 0

        lax.fori_loop(0, _C, zrow, 0)

        def zcopy(kk, _):
            pltpu.sync_copy(
                mb.at[0],
                aggr.at[pl.ds(s * _RPT + kk * _ZC, _ZC)],
            )
            return 0

        lax.fori_loop(0, nz, zcopy, 0)
        plsc.subcore_barrier()

        def start_idx(g):
            pltpu.async_copy(ids_hbm.at[tid, g], idr.at[lax.rem(g, 4)],
                             isem.at[lax.rem(g, 4)])

        def wait_idx(g):
            pltpu.make_async_copy(
                ids_hbm.at[0, 0], idr.at[lax.rem(g, 4)],
                isem.at[lax.rem(g, 4)]
            ).wait()

        def start_in(g, b):
            pltpu.async_copy(x_hbm.at[idr.at[lax.rem(g, 4), 0]], xg.at[b],
                             gsem.at[b])
            pltpu.async_copy(e_hbm.at[pl.ds(ebase + g * _C, _C)], eb.at[b],
                             esem.at[b])

        def wait_in(b):
            pltpu.make_async_copy(
                x_hbm.at[pl.ds(0, _C)], xg.at[b], gsem.at[b]
            ).wait()
            pltpu.make_async_copy(
                e_hbm.at[pl.ds(0, _C)], eb.at[b], esem.at[b]
            ).wait()

        def wait_scatter(b):
            pltpu.make_async_copy(
                mb.at[b], aggr.at[pl.ds(0, _C)], ssem.at[b]
            ).wait()

        for q in range(3):
            start_idx(q)
        wait_idx(0)
        start_in(0, 0)

        def step(i, _):
            for b in range(2):
                g = i * 2 + b

                @pl.when(g + 3 < _NCH)
                def _():
                    start_idx(g + 3)

                @pl.when(g + 1 < _NCH)
                def _():
                    wait_idx(g + 1)
                    start_in(g + 1, 1 - b)

                @pl.when(g >= 2)
                def _():
                    wait_scatter(b)

                wait_in(b)

                def row(r, _):
                    for rr in range(2):
                        rw = 2 * r + rr
                        for j in range(_D // 32):
                            sl = pl.ds(j * 32, 32)
                            v = xg[b, rw, sl] + eb[b, rw, sl]
                            v = jnp.maximum(v, jnp.bfloat16(0))
                            u = plsc.bitcast(v, jnp.int32)
                            ev = plsc.bitcast(u << 16, jnp.float32)
                            od = plsc.bitcast(u & jnp.int32(-65536),
                                              jnp.float32)
                            mb[b, rw, pl.ds(j * 32, 16)] = ev
                            mb[b, rw, pl.ds(j * 32 + 16, 16)] = od
                    return 0

                lax.fori_loop(0, _C // 2, row, 0)

                pltpu.async_copy(mb.at[b], aggr.at[idr.at[lax.rem(g, 4), 1]],
                                 ssem.at[b], add=True)
            return 0

        lax.fori_loop(0, _NCH // 2, step, 0)

        for b in range(2):
            wait_scatter(b)
        plsc.subcore_barrier()

        def wb(kk, _):
            pltpu.sync_copy(
                aggr.at[pl.ds(s * _RPT + kk * _ZC, _ZC)],
                out_hbm.at[c, pl.ds(s * _RPT + kk * _ZC, _ZC)],
            )
            return 0

        lax.fori_loop(0, nz, wb, 0)

    return k(x, ids, e)


def _tc_mlp(x, parts, W1, b1, W2, b2):
    R = 1000

    def body(x_ref, p_ref, w1_ref, b1_ref, w2_ref, b2_ref, o_ref):
        h = x_ref[...] + p_ref[0] + p_ref[1]
        t = jnp.maximum(
            jnp.dot(h, w1_ref[...], preferred_element_type=jnp.float32)
            + b1_ref[...],
            0.0,
        )
        o_ref[...] = (
            jnp.dot(t, w2_ref[...], preferred_element_type=jnp.float32)
            + b2_ref[...]
        )

    return pl.pallas_call(
        body,
        grid=(_N // R,),
        in_specs=[
            pl.BlockSpec((R, _D), lambda i: (i, 0)),
            pl.BlockSpec((_NC, R, _D), lambda i: (0, i, 0)),
            pl.BlockSpec((_D, _D), lambda i: (0, 0)),
            pl.BlockSpec((1, _D), lambda i: (0, 0)),
            pl.BlockSpec((_D, _D), lambda i: (0, 0)),
            pl.BlockSpec((1, _D), lambda i: (0, 0)),
        ],
        out_specs=pl.BlockSpec((R, _D), lambda i: (i, 0)),
        out_shape=jax.ShapeDtypeStruct((_N, _D), jnp.float32),
    )(x, parts, W1, b1.reshape(1, _D), W2, b2.reshape(1, _D))


def kernel(x, edge_index, edge_attr, W_edge, b_edge, W1, b1, W2, b2):
    ids = jnp.stack(
        [edge_index[0].reshape(_NT, _NCH, _C),
         edge_index[1].reshape(_NT, _NCH, _C)],
        axis=2,
    )
    xb = x.astype(jnp.bfloat16)
    e = _tc_edge_embed(edge_attr, W_edge, b_edge)
    parts = _sc_edge_aggr(xb, ids, e)
    return _tc_mlp(x[:, _PERM], parts, W1[_PERM, :], b1, W2, b2)


# speedup vs baseline: 1.6926x; 1.6926x over previous
import functools

import jax
import jax.numpy as jnp
from jax import lax
from jax.experimental import pallas as pl
from jax.experimental.pallas import tpu as pltpu
from jax.experimental.pallas import tpu_sc as plsc

_N = 10000
_E = 320000
_D = 128
_DH = 64
_DE = 16
_NC = 2
_NS = 16
_NT = _NC * _NS
_EPT = _E // _NT
_C = 40
_NCH = _EPT // _C
_RPT = 640
_ZC = 40


def _tc_edge_embed(edge_attr, W_edge, b_edge):
    B = 3200

    def rne_hi16(v):
        b = lax.bitcast_convert_type(v, jnp.int32)
        return b + 0x7FFF + ((b >> 16) & 1)

    def body(a_ref, w_ref, b_ref, o_ref):
        acc = (
            jnp.dot(a_ref[...], w_ref[...], preferred_element_type=jnp.float32)
            + b_ref[...]
        )
        lo = rne_hi16(acc[:, :_DH])
        hi = rne_hi16(acc[:, _DH:])
        o_ref[...] = ((lo >> 16) & 0xFFFF) | (hi & jnp.int32(-65536))

    return pl.pallas_call(
        body,
        grid=(_E // B,),
        in_specs=[
            pl.BlockSpec((B, _DE), lambda i: (i, 0)),
            pl.BlockSpec((_DE, _D), lambda i: (0, 0)),
            pl.BlockSpec((1, _D), lambda i: (0, 0)),
        ],
        out_specs=pl.BlockSpec((B, _DH), lambda i: (i, 0)),
        out_shape=jax.ShapeDtypeStruct((_E, _DH), jnp.int32),
    )(edge_attr, W_edge, b_edge.reshape(1, _D))


def _sc_edge_aggr(x, ids, ep):
    mesh = plsc.VectorSubcoreMesh(core_axis_name="c", subcore_axis_name="s")

    @functools.partial(
        pl.kernel,
        out_type=jax.ShapeDtypeStruct((_NC, _N, _D), jnp.float32),
        mesh=mesh,
        scratch_types=[
            pltpu.VMEM_SHARED((_N, _D), jnp.float32),
            pltpu.VMEM((4, 2, _C), jnp.int32),
            pltpu.VMEM((2, _C, _D), jnp.float32),
            pltpu.VMEM((2, _C, _DH), jnp.int32),
            pltpu.VMEM((2, _C, _D), jnp.float32),
            pltpu.SemaphoreType.DMA((4,)),
            pltpu.SemaphoreType.DMA((2,)),
            pltpu.SemaphoreType.DMA((2,)),
            pltpu.SemaphoreType.DMA((2,)),
        ],
        compiler_params=pltpu.CompilerParams(needs_layout_passes=False),
    )
    def k(x_hbm, ids_hbm, e_hbm, out_hbm,
          aggr, idr, xg, eb, mb, isem, gsem, esem, ssem):
        c = lax.axis_index("c")
        s = lax.axis_index("s")
        tid = c * _NS + s
        ebase = tid * _EPT

        nz = jnp.where(s == _NS - 1, (_N - (_NS - 1) * _RPT) // _ZC, _RPT // _ZC)

        def zrow(i, _):
            for j in range(_D // 16):
                mb[0, i, pl.ds(j * 16, 16)] = jnp.zeros((16,), jnp.float32)
            return 0

        lax.fori_loop(0, _C, zrow, 0)

        def zcopy(kk, _):
            pltpu.sync_copy(
                mb.at[0],
                aggr.at[pl.ds(s * _RPT + kk * _ZC, _ZC)],
            )
            return 0

        lax.fori_loop(0, nz, zcopy, 0)
        plsc.subcore_barrier()

        def start_idx(g):
            pltpu.async_copy(ids_hbm.at[tid, g], idr.at[lax.rem(g, 4)],
                             isem.at[lax.rem(g, 4)])

        def wait_idx(g):
            pltpu.make_async_copy(
                ids_hbm.at[0, 0], idr.at[lax.rem(g, 4)],
                isem.at[lax.rem(g, 4)]
            ).wait()

        def start_in(g, b):
            pltpu.async_copy(x_hbm.at[idr.at[lax.rem(g, 4), 0]], xg.at[b],
                             gsem.at[b])
            pltpu.async_copy(e_hbm.at[pl.ds(ebase + g * _C, _C)], eb.at[b],
                             esem.at[b])

        def wait_in(b):
            pltpu.make_async_copy(
                x_hbm.at[pl.ds(0, _C)], xg.at[b], gsem.at[b]
            ).wait()
            pltpu.make_async_copy(
                e_hbm.at[pl.ds(0, _C)], eb.at[b], esem.at[b]
            ).wait()

        def wait_scatter(b):
            pltpu.make_async_copy(
                mb.at[b], aggr.at[pl.ds(0, _C)], ssem.at[b]
            ).wait()

        for q in range(3):
            start_idx(q)
        wait_idx(0)
        start_in(0, 0)

        def step(i, _):
            for b in range(2):
                g = i * 2 + b

                @pl.when(g + 3 < _NCH)
                def _():
                    start_idx(g + 3)

                @pl.when(g + 1 < _NCH)
                def _():
                    wait_idx(g + 1)
                    start_in(g + 1, 1 - b)

                @pl.when(g >= 2)
                def _():
                    wait_scatter(b)

                wait_in(b)

                def row(r, _):
                    for rr in range(2):
                        rw = 2 * r + rr
                        for j in range(_DH // 16):
                            ue = eb[b, rw, pl.ds(j * 16, 16)]
                            el = plsc.bitcast(ue << 16, jnp.float32)
                            eh = plsc.bitcast(ue & jnp.int32(-65536),
                                              jnp.float32)
                            xl = xg[b, rw, pl.ds(j * 16, 16)]
                            xh = xg[b, rw, pl.ds(_DH + j * 16, 16)]
                            mb[b, rw, pl.ds(j * 16, 16)] = jnp.maximum(
                                xl + el, 0.0)
                            mb[b, rw, pl.ds(_DH + j * 16, 16)] = jnp.maximum(
                                xh + eh, 0.0)
                    return 0

                lax.fori_loop(0, _C // 2, row, 0)

                pltpu.async_copy(mb.at[b], aggr.at[idr.at[lax.rem(g, 4), 1]],
                                 ssem.at[b], add=True)
            return 0

        lax.fori_loop(0, _NCH // 2, step, 0)

        for b in range(2):
            wait_scatter(b)
        plsc.subcore_barrier()

        def wb(kk, _):
            pltpu.sync_copy(
                aggr.at[pl.ds(s * _RPT + kk * _ZC, _ZC)],
                out_hbm.at[c, pl.ds(s * _RPT + kk * _ZC, _ZC)],
            )
            return 0

        lax.fori_loop(0, nz, wb, 0)

    return k(x, ids, ep)


def _tc_mlp(x, parts, W1, b1, W2, b2):
    R = 1000

    def body(x_ref, p_ref, w1_ref, b1_ref, w2_ref, b2_ref, o_ref):
        h = x_ref[...] + p_ref[0] + p_ref[1]
        t = jnp.maximum(
            jnp.dot(h, w1_ref[...], preferred_element_type=jnp.float32)
            + b1_ref[...],
            0.0,
        )
        o_ref[...] = (
            jnp.dot(t, w2_ref[...], preferred_element_type=jnp.float32)
            + b2_ref[...]
        )

    return pl.pallas_call(
        body,
        grid=(_N // R,),
        in_specs=[
            pl.BlockSpec((R, _D), lambda i: (i, 0)),
            pl.BlockSpec((_NC, R, _D), lambda i: (0, i, 0)),
            pl.BlockSpec((_D, _D), lambda i: (0, 0)),
            pl.BlockSpec((1, _D), lambda i: (0, 0)),
            pl.BlockSpec((_D, _D), lambda i: (0, 0)),
            pl.BlockSpec((1, _D), lambda i: (0, 0)),
        ],
        out_specs=pl.BlockSpec((R, _D), lambda i: (i, 0)),
        out_shape=jax.ShapeDtypeStruct((_N, _D), jnp.float32),
    )(x, parts, W1, b1.reshape(1, _D), W2, b2.reshape(1, _D))


def kernel(x, edge_index, edge_attr, W_edge, b_edge, W1, b1, W2, b2):
    ids = jnp.stack(
        [edge_index[0].reshape(_NT, _NCH, _C),
         edge_index[1].reshape(_NT, _NCH, _C)],
        axis=2,
    )
    ep = _tc_edge_embed(edge_attr, W_edge, b_edge)
    parts = _sc_edge_aggr(x, ids, ep)
    return _tc_mlp(x, parts, W1, b1, W2, b2)


# speedup vs baseline: 2.4757x; 1.4627x over previous
import functools

import jax
import jax.numpy as jnp
from jax import lax
from jax.experimental import pallas as pl
from jax.experimental.pallas import tpu as pltpu
from jax.experimental.pallas import tpu_sc as plsc

_N = 10000
_E = 320000
_D = 128
_DH = 64
_DE = 16
_NC = 2
_NS = 16
_NT = _NC * _NS
_EPT = _E // _NT
_C = 40
_NCH = _EPT // _C
_RPT = 640
_ZC = 40


def _tc_edge_embed(edge_attr_t, W_edge, b_edge):
    B = 3200

    def rne_hi16(v):
        b = lax.bitcast_convert_type(v, jnp.int32)
        return b + 0x7FFF + ((b >> 16) & 1)

    def body(a_ref, w_ref, b_ref, o_ref):
        acc = (
            lax.dot_general(
                a_ref[...], w_ref[...],
                dimension_numbers=(((0,), (0,)), ((), ())),
                preferred_element_type=jnp.float32,
            )
            + b_ref[...]
        )
        lo = rne_hi16(acc[:, :_DH])
        hi = rne_hi16(acc[:, _DH:])
        o_ref[...] = ((lo >> 16) & 0xFFFF) | (hi & jnp.int32(-65536))

    return pl.pallas_call(
        body,
        grid=(_E // B,),
        in_specs=[
            pl.BlockSpec((_DE, B), lambda i: (0, i)),
            pl.BlockSpec((_DE, _D), lambda i: (0, 0)),
            pl.BlockSpec((1, _D), lambda i: (0, 0)),
        ],
        out_specs=pl.BlockSpec((B, _DH), lambda i: (i, 0)),
        out_shape=jax.ShapeDtypeStruct((_E, _DH), jnp.int32),
    )(edge_attr_t, W_edge, b_edge.reshape(1, _D))


def _sc_edge_aggr(x, eidx, ep):
    mesh = plsc.VectorSubcoreMesh(core_axis_name="c", subcore_axis_name="s")

    @functools.partial(
        pl.kernel,
        out_type=jax.ShapeDtypeStruct((_NC, _N, _D), jnp.float32),
        mesh=mesh,
        scratch_types=[
            pltpu.VMEM_SHARED((_N, _D), jnp.float32),
            pltpu.VMEM((4, 2, _C), jnp.int32),
            pltpu.VMEM((2, _C, _D), jnp.float32),
            pltpu.VMEM((2, _C, _DH), jnp.int32),
            pltpu.VMEM((2, _C, _D), jnp.float32),
            pltpu.SemaphoreType.DMA((4,)),
            pltpu.SemaphoreType.DMA((2,)),
            pltpu.SemaphoreType.DMA((2,)),
            pltpu.SemaphoreType.DMA((2,)),
        ],
        compiler_params=pltpu.CompilerParams(needs_layout_passes=False),
    )
    def k(x_hbm, eidx_hbm, e_hbm, out_hbm,
          aggr, idr, xg, eb, mb, isem, gsem, esem, ssem):
        c = lax.axis_index("c")
        s = lax.axis_index("s")
        tid = c * _NS + s
        ebase = tid * _EPT

        nz = jnp.where(s == _NS - 1, (_N - (_NS - 1) * _RPT) // _ZC, _RPT // _ZC)

        def zrow(i, _):
            for j in range(_D // 16):
                mb[0, i, pl.ds(j * 16, 16)] = jnp.zeros((16,), jnp.float32)
            return 0

        lax.fori_loop(0, _C, zrow, 0)

        def zcopy(kk, _):
            pltpu.sync_copy(
                mb.at[0],
                aggr.at[pl.ds(s * _RPT + kk * _ZC, _ZC)],
            )
            return 0

        lax.fori_loop(0, nz, zcopy, 0)
        plsc.subcore_barrier()

        def start_idx(g):
            q = lax.rem(g, 4)
            off = ebase + g * _C
            pltpu.async_copy(eidx_hbm.at[pl.ds(off, _C)], idr.at[q, 0],
                             isem.at[q])
            pltpu.async_copy(eidx_hbm.at[pl.ds(_E + off, _C)], idr.at[q, 1],
                             isem.at[q])

        def wait_idx(g):
            q = lax.rem(g, 4)
            for r in range(2):
                pltpu.make_async_copy(
                    eidx_hbm.at[pl.ds(0, _C)], idr.at[q, r], isem.at[q]
                ).wait()

        def start_in(g, b):
            pltpu.async_copy(x_hbm.at[idr.at[lax.rem(g, 4), 0]], xg.at[b],
                             gsem.at[b])
            pltpu.async_copy(e_hbm.at[pl.ds(ebase + g * _C, _C)], eb.at[b],
                             esem.at[b])

        def wait_in(b):
            pltpu.make_async_copy(
                x_hbm.at[pl.ds(0, _C)], xg.at[b], gsem.at[b]
            ).wait()
            pltpu.make_async_copy(
                e_hbm.at[pl.ds(0, _C)], eb.at[b], esem.at[b]
            ).wait()

        def wait_scatter(b):
            pltpu.make_async_copy(
                mb.at[b], aggr.at[pl.ds(0, _C)], ssem.at[b]
            ).wait()

        for q in range(3):
            start_idx(q)
        wait_idx(0)
        start_in(0, 0)

        def step(i, _):
            for b in range(2):
                g = i * 2 + b

                @pl.when(g + 3 < _NCH)
                def _():
                    start_idx(g + 3)

                @pl.when(g + 1 < _NCH)
                def _():
                    wait_idx(g + 1)
                    start_in(g + 1, 1 - b)

                @pl.when(g >= 2)
                def _():
                    wait_scatter(b)

                wait_in(b)

                def row(r, _):
                    for rr in range(2):
                        rw = 2 * r + rr
                        for j in range(_DH // 16):
                            ue = eb[b, rw, pl.ds(j * 16, 16)]
                            el = plsc.bitcast(ue << 16, jnp.float32)
                            eh = plsc.bitcast(ue & jnp.int32(-65536),
                                              jnp.float32)
                            xl = xg[b, rw, pl.ds(j * 16, 16)]
                            xh = xg[b, rw, pl.ds(_DH + j * 16, 16)]
                            mb[b, rw, pl.ds(j * 16, 16)] = jnp.maximum(
                                xl + el, 0.0)
                            mb[b, rw, pl.ds(_DH + j * 16, 16)] = jnp.maximum(
                                xh + eh, 0.0)
                    return 0

                lax.fori_loop(0, _C // 2, row, 0)

                pltpu.async_copy(mb.at[b], aggr.at[idr.at[lax.rem(g, 4), 1]],
                                 ssem.at[b], add=True)
            return 0

        lax.fori_loop(0, _NCH // 2, step, 0)

        for b in range(2):
            wait_scatter(b)
        plsc.subcore_barrier()

        def wb(kk, _):
            pltpu.sync_copy(
                aggr.at[pl.ds(s * _RPT + kk * _ZC, _ZC)],
                out_hbm.at[c, pl.ds(s * _RPT + kk * _ZC, _ZC)],
            )
            return 0

        lax.fori_loop(0, nz, wb, 0)

    return k(x, eidx, ep)


def _tc_mlp(x, parts, W1, b1, W2, b2):
    R = 1000

    def body(x_ref, p_ref, w1_ref, b1_ref, w2_ref, b2_ref, o_ref):
        h = x_ref[...] + p_ref[0] + p_ref[1]
        t = jnp.maximum(
            jnp.dot(h, w1_ref[...], preferred_element_type=jnp.float32)
            + b1_ref[...],
            0.0,
        )
        o_ref[...] = (
            jnp.dot(t, w2_ref[...], preferred_element_type=jnp.float32)
            + b2_ref[...]
        )

    return pl.pallas_call(
        body,
        grid=(_N // R,),
        in_specs=[
            pl.BlockSpec((R, _D), lambda i: (i, 0)),
            pl.BlockSpec((_NC, R, _D), lambda i: (0, i, 0)),
            pl.BlockSpec((_D, _D), lambda i: (0, 0)),
            pl.BlockSpec((1, _D), lambda i: (0, 0)),
            pl.BlockSpec((_D, _D), lambda i: (0, 0)),
            pl.BlockSpec((1, _D), lambda i: (0, 0)),
        ],
        out_specs=pl.BlockSpec((R, _D), lambda i: (i, 0)),
        out_shape=jax.ShapeDtypeStruct((_N, _D), jnp.float32),
    )(x, parts, W1, b1.reshape(1, _D), W2, b2.reshape(1, _D))


def kernel(x, edge_index, edge_attr, W_edge, b_edge, W1, b1, W2, b2):
    eidx = edge_index.reshape(-1)
    ep = _tc_edge_embed(edge_attr.T, W_edge, b_edge)
    parts = _sc_edge_aggr(x, eidx, ep)
    return _tc_mlp(x, parts, W1, b1, W2, b2)


# speedup vs baseline: 2.4783x; 1.0010x over previous
import functools

import jax
import jax.numpy as jnp
import numpy as np
from jax import lax
from jax.experimental import pallas as pl
from jax.experimental.pallas import tpu as pltpu
from jax.experimental.pallas import tpu_sc as plsc

_N = 10000
_E = 320000
_D = 128
_DH = 64
_DE = 16
_NC = 2
_NS = 16
_NT = _NC * _NS
_EPT = _E // _NT
_C = 40
_NCH = _EPT // _C
_RPT = 640
_ZC = 40


def _tc_edge_embed(edge_attr_t, W_edge, b_edge):
    B = 3200

    def rne_hi16(v):
        b = lax.bitcast_convert_type(v, jnp.int32)
        return b + 0x7FFF + ((b >> 16) & 1)

    def body(a_ref, w_ref, b_ref, o_ref):
        acc = (
            lax.dot_general(
                a_ref[...], w_ref[...],
                dimension_numbers=(((0,), (0,)), ((), ())),
                preferred_element_type=jnp.float32,
            )
            + b_ref[...]
        )
        lo = rne_hi16(acc[:, :_DH])
        hi = rne_hi16(acc[:, _DH:])
        o_ref[...] = ((lo >> 16) & 0xFFFF) | (hi & jnp.int32(-65536))

    return pl.pallas_call(
        body,
        grid=(_E // B,),
        in_specs=[
            pl.BlockSpec((_DE, B), lambda i: (0, i)),
            pl.BlockSpec((_DE, _D), lambda i: (0, 0)),
            pl.BlockSpec((1, _D), lambda i: (0, 0)),
        ],
        out_specs=pl.BlockSpec((B, _DH), lambda i: (i, 0)),
        out_shape=jax.ShapeDtypeStruct((_E, _DH), jnp.int32),
    )(edge_attr_t, W_edge, b_edge.reshape(1, _D))


def _sc_edge_aggr(x, eidx, ep):
    mesh = plsc.VectorSubcoreMesh(core_axis_name="c", subcore_axis_name="s")

    @functools.partial(
        pl.kernel,
        out_type=jax.ShapeDtypeStruct((_NC, _N, _D), jnp.float32),
        mesh=mesh,
        scratch_types=[
            pltpu.VMEM_SHARED((_N, _D), jnp.float32),
            pltpu.VMEM((4, 2, _C), jnp.int32),
            pltpu.VMEM((2, _C, _D), jnp.float32),
            pltpu.VMEM((2, _C, _DH), jnp.int32),
            pltpu.VMEM((2, _C, _D), jnp.float32),
            pltpu.SemaphoreType.DMA((4,)),
            pltpu.SemaphoreType.DMA((2,)),
            pltpu.SemaphoreType.DMA((2,)),
            pltpu.SemaphoreType.DMA((2,)),
        ],
        compiler_params=pltpu.CompilerParams(needs_layout_passes=False),
    )
    def k(x_hbm, eidx_hbm, e_hbm, out_hbm,
          aggr, idr, xg, eb, mb, isem, gsem, esem, ssem):
        c = lax.axis_index("c")
        s = lax.axis_index("s")
        tid = c * _NS + s
        ebase = tid * _EPT

        nz = jnp.where(s == _NS - 1, (_N - (_NS - 1) * _RPT) // _ZC, _RPT // _ZC)

        def zrow(i, _):
            for j in range(_D // 16):
                mb[0, i, pl.ds(j * 16, 16)] = jnp.zeros((16,), jnp.float32)
            return 0

        lax.fori_loop(0, _C, zrow, 0)

        def zcopy(kk, _):
            pltpu.sync_copy(
                mb.at[0],
                aggr.at[pl.ds(s * _RPT + kk * _ZC, _ZC)],
            )
            return 0

        lax.fori_loop(0, nz, zcopy, 0)
        plsc.subcore_barrier()

        def start_idx(g):
            q = lax.rem(g, 4)
            off = ebase + g * _C
            pltpu.async_copy(eidx_hbm.at[pl.ds(off, _C)], idr.at[q, 0],
                             isem.at[q])
            pltpu.async_copy(eidx_hbm.at[pl.ds(_E + off, _C)], idr.at[q, 1],
                             isem.at[q])

        def wait_idx(g):
            q = lax.rem(g, 4)
            for r in range(2):
                pltpu.make_async_copy(
                    eidx_hbm.at[pl.ds(0, _C)], idr.at[q, r], isem.at[q]
                ).wait()

        def start_in(g, b):
            pltpu.async_copy(x_hbm.at[idr.at[lax.rem(g, 4), 0]], xg.at[b],
                             gsem.at[b])
            pltpu.async_copy(e_hbm.at[pl.ds(ebase + g * _C, _C)], eb.at[b],
                             esem.at[b])

        def wait_in(b):
            pltpu.make_async_copy(
                x_hbm.at[pl.ds(0, _C)], xg.at[b], gsem.at[b]
            ).wait()
            pltpu.make_async_copy(
                e_hbm.at[pl.ds(0, _C)], eb.at[b], esem.at[b]
            ).wait()

        def wait_scatter(b):
            pltpu.make_async_copy(
                mb.at[b], aggr.at[pl.ds(0, _C)], ssem.at[b]
            ).wait()

        for q in range(3):
            start_idx(q)
        wait_idx(0)
        start_in(0, 0)

        def step(i, _):
            for b in range(2):
                g = i * 2 + b

                @pl.when(g + 3 < _NCH)
                def _():
                    start_idx(g + 3)

                @pl.when(g + 1 < _NCH)
                def _():
                    wait_idx(g + 1)
                    start_in(g + 1, 1 - b)

                @pl.when(g >= 2)
                def _():
                    wait_scatter(b)

                wait_in(b)

                def row(r, _):
                    for rr in range(2):
                        rw = 2 * r + rr
                        for j in range(_DH // 16):
                            ue = eb[b, rw, pl.ds(j * 16, 16)]
                            el = plsc.bitcast(ue << 16, jnp.float32)
                            eh = plsc.bitcast(ue & jnp.int32(-65536),
                                              jnp.float32)
                            xl = xg[b, rw, pl.ds(j * 16, 16)]
                            xh = xg[b, rw, pl.ds(_DH + j * 16, 16)]
                            mb[b, rw, pl.ds(j * 16, 16)] = jnp.maximum(
                                xl + el, 0.0)
                            mb[b, rw, pl.ds(_DH + j * 16, 16)] = jnp.maximum(
                                xh + eh, 0.0)
                    return 0

                lax.fori_loop(0, _C // 2, row, 0)

                pltpu.async_copy(mb.at[b], aggr.at[idr.at[lax.rem(g, 4), 1]],
                                 ssem.at[b], add=True)
            return 0

        lax.fori_loop(0, _NCH // 2, step, 0)

        for b in range(2):
            wait_scatter(b)
        plsc.subcore_barrier()

        def wb(kk, _):
            pltpu.sync_copy(
                aggr.at[pl.ds(s * _RPT + kk * _ZC, _ZC)],
                out_hbm.at[c, pl.ds(s * _RPT + kk * _ZC, _ZC)],
            )
            return 0

        lax.fori_loop(0, nz, wb, 0)

    return k(x, eidx, ep)


def _tc_mlp(x, parts, W1, b1, W2, b2):
    R = 1000

    def body(x_ref, p_ref, w1_ref, b1_ref, w2_ref, b2_ref, o_ref):
        h = x_ref[...] + p_ref[0] + p_ref[1]
        t = jnp.maximum(
            jnp.dot(h, w1_ref[...], preferred_element_type=jnp.float32)
            + b1_ref[...],
            0.0,
        )
        o_ref[...] = (
            jnp.dot(t, w2_ref[...], preferred_element_type=jnp.float32)
            + b2_ref[...]
        )

    return pl.pallas_call(
        body,
        grid=(_N // R,),
        in_specs=[
            pl.BlockSpec((R, _D), lambda i: (i, 0)),
            pl.BlockSpec((_NC, R, _D), lambda i: (0, i, 0)),
            pl.BlockSpec((_D, _D), lambda i: (0, 0)),
            pl.BlockSpec((1, _D), lambda i: (0, 0)),
            pl.BlockSpec((_D, _D), lambda i: (0, 0)),
            pl.BlockSpec((1, _D), lambda i: (0, 0)),
        ],
        out_specs=pl.BlockSpec((R, _D), lambda i: (i, 0)),
        out_shape=jax.ShapeDtypeStruct((_N, _D), jnp.float32),
    )(x, parts, W1, b1.reshape(1, _D), W2, b2.reshape(1, _D))


def kernel(x, edge_index, edge_attr, W_edge, b_edge, W1, b1, W2, b2):
    eidx = edge_index.reshape(-1)
    ep = _tc_edge_embed(edge_attr.T, W_edge, b_edge)
    parts = _sc_edge_aggr(x, eidx, ep)
    return _tc_mlp(x, parts, W1, b1, W2, b2)
